# baseline (device time: 146638 ns/iter reference)
import jax
import jax.numpy as jnp
from jax import lax
from jax.experimental import pallas as pl
from jax.experimental.pallas import tpu as pltpu

N_DEV = 4
M = 4096
N = 2048
NH = N // 2
CC = 2
CH = NH // CC
KS = 1024
MQ = M // N_DEV
CW, CCW = 0, 1


def _body(x_hbm, w_ref, yc_ref, scale_ref,
          xs, xb, rcv_cw, rcv_ccw, sb_cw, sb_ccw, own_cw, own_ccw, amax_ref,
          x_sem, rs_s_cw, rs_r_cw, rs_s_ccw, rs_r_ccw,
          ax_s, ax_r, ag_s_cw, ag_r_cw, ag_s_ccw, ag_r_ccw):
    my = lax.axis_index("i")
    right = lax.rem(my + 1, N_DEV)
    left = lax.rem(my + N_DEV - 1, N_DEV)

    def x_dma(b, slot):
        q = lax.rem(my - b + 2 * N_DEV, N_DEV)
        return pltpu.make_async_copy(
            x_hbm.at[pl.ds(q * MQ, MQ), :], xs.at[slot], x_sem.at[slot])

    dma0 = x_dma(0, 0)
    dma0.start()
    dma1 = x_dma(1, 1)
    dma1.start()

    barrier = pltpu.get_barrier_semaphore()
    for nbr in (left, right):
        pl.semaphore_signal(barrier, inc=1, device_id=(nbr,),
                            device_id_type=pl.DeviceIdType.MESH)
    pl.semaphore_wait(barrier, 2)

    dma0.wait()
    xb[0] = xs[0].astype(jnp.bfloat16)
    dma3 = x_dma(3, 0)
    dma3.start()

    def wslice(ring, c):
        base = ring * NH + c * CH
        return w_ref[:, base:base + CH]

    def rs_rdma(s, ring, c):
        rcv = rcv_cw if ring == CW else rcv_ccw
        sb = sb_cw if ring == CW else sb_ccw
        return pltpu.make_async_remote_copy(
            src_ref=sb.at[c] if s == 0 else rcv.at[s - 1, c],
            dst_ref=rcv.at[s, c],
            send_sem=(rs_s_cw if ring == CW else rs_s_ccw).at[s, c],
            recv_sem=(rs_r_cw if ring == CW else rs_r_ccw).at[s, c],
            device_id=(right,) if ring == CW else (left,),
            device_id_type=pl.DeviceIdType.MESH,
        )

    cur = {}
    for c in range(CC):
        sb_cw[c] = jnp.dot(xb[0], wslice(CW, c),
                           preferred_element_type=jnp.float32
                           ).astype(jnp.bfloat16)
        sb_ccw[c] = jnp.dot(xb[0], wslice(CCW, c),
                            preferred_element_type=jnp.float32
                            ).astype(jnp.bfloat16)
        for ring in (CW, CCW):
            d = rs_rdma(0, ring, c)
            d.start()
            cur[(ring, c)] = d

    am_parts = []
    dma2 = None
    for s in range(N_DEV - 1):
        if s == 0:
            dma1.wait()
            xb[1] = xs[1].astype(jnp.bfloat16)
            dma2 = x_dma(2, 1)
            dma2.start()
            dma3.wait()
            xb[3] = xs[0].astype(jnp.bfloat16)
        elif s == 1:
            dma2.wait()
            xb[2] = xs[1].astype(jnp.bfloat16)
        nxt = {}
        for c in range(CC):
            for ring in (CW, CCW):
                blk = s + 1 if ring == CW else 3 - s
                part = jnp.dot(xb[blk], wslice(ring, c),
                               preferred_element_type=jnp.float32)
                cur[(ring, c)].wait()
                rcv = rcv_cw if ring == CW else rcv_ccw
                tot = rcv[s, c].astype(jnp.float32) + part
                if s < N_DEV - 2:
                    rcv[s, c] = tot.astype(jnp.bfloat16)
                    d = rs_rdma(s + 1, ring, c)
                    d.start()
                    nxt[(ring, c)] = d
                else:
                    own = own_cw if ring == CW else own_ccw
                    own[c] = tot
                    am_parts.append(jnp.max(tot))
        cur = nxt

    am = am_parts[0]
    for p in am_parts[1:]:
        am = jnp.maximum(am, p)
    am = jnp.maximum(am, 0.0)
    amax_ref[0, :, :] = jnp.full((8, 128), am, dtype=jnp.float32)
    opp = lax.rem(my + 2, N_DEV)
    ax_rdmas = []
    for j, (tgt, slot) in enumerate(((right, 3), (left, 1), (opp, 2))):
        rdma = pltpu.make_async_remote_copy(
            src_ref=amax_ref.at[0], dst_ref=amax_ref.at[slot],
            send_sem=ax_s.at[j], recv_sem=ax_r.at[slot - 1],
            device_id=(tgt,), device_id_type=pl.DeviceIdType.MESH)
        rdma.start()
        ax_rdmas.append(rdma)
    for rdma in ax_rdmas:
        rdma.wait()
    g_amax = jnp.max(amax_ref[:, :, :])
    scale = g_amax / 127.0
    inv_scale = 127.0 / g_amax
    scale_ref[0, 0] = scale

    qcw_own = lax.rem(my + 1, N_DEV)
    qccw_own = lax.rem(my + N_DEV - 1, N_DEV)
    for c in range(CC):
        qcw = jnp.clip(jnp.round(jnp.maximum(own_cw[c], 0.0) * inv_scale),
                       0.0, 127.0)
        qccw = jnp.clip(jnp.round(jnp.maximum(own_ccw[c], 0.0) * inv_scale),
                        0.0, 127.0)
        yc_ref[pl.ds(qcw_own * MQ, MQ),
               c * CH:(c + 1) * CH] = qcw.astype(jnp.int8)
        yc_ref[pl.ds(qccw_own * MQ, MQ),
               NH + c * CH:NH + (c + 1) * CH] = qccw.astype(jnp.int8)
    for t in range(N_DEV - 1):
        qcw_s = lax.rem(my + 1 - t + N_DEV, N_DEV)
        qccw_s = lax.rem(my + N_DEV - 1 + t, N_DEV)
        cw_slc = yc_ref.at[pl.ds(qcw_s * MQ, MQ), 0:NH]
        ccw_slc = yc_ref.at[pl.ds(qccw_s * MQ, MQ), NH:N]
        cw = pltpu.make_async_remote_copy(
            src_ref=cw_slc, dst_ref=cw_slc,
            send_sem=ag_s_cw.at[t], recv_sem=ag_r_cw.at[t],
            device_id=(right,), device_id_type=pl.DeviceIdType.MESH)
        ccw = pltpu.make_async_remote_copy(
            src_ref=ccw_slc, dst_ref=ccw_slc,
            send_sem=ag_s_ccw.at[t], recv_sem=ag_r_ccw.at[t],
            device_id=(left,), device_id_type=pl.DeviceIdType.MESH)
        cw.start()
        ccw.start()
        cw.wait()
        ccw.wait()


def kernel(x, w_mat):
    w16 = w_mat.astype(jnp.bfloat16)

    yc, scale = pl.pallas_call(
        _body,
        out_shape=[
            jax.ShapeDtypeStruct((M, N), jnp.int8),
            jax.ShapeDtypeStruct((1, 1), jnp.float32),
        ],
        in_specs=[
            pl.BlockSpec(memory_space=pl.ANY),
            pl.BlockSpec(memory_space=pltpu.VMEM),
        ],
        out_specs=[
            pl.BlockSpec(memory_space=pltpu.VMEM),
            pl.BlockSpec(memory_space=pltpu.SMEM),
        ],
        scratch_shapes=[
            pltpu.VMEM((2, MQ, KS), jnp.float32),
            pltpu.VMEM((N_DEV, MQ, KS), jnp.bfloat16),
            pltpu.VMEM((N_DEV - 1, CC, MQ, CH), jnp.bfloat16),
            pltpu.VMEM((N_DEV - 1, CC, MQ, CH), jnp.bfloat16),
            pltpu.VMEM((CC, MQ, CH), jnp.bfloat16),
            pltpu.VMEM((CC, MQ, CH), jnp.bfloat16),
            pltpu.VMEM((CC, MQ, CH), jnp.float32),
            pltpu.VMEM((CC, MQ, CH), jnp.float32),
            pltpu.VMEM((N_DEV, 8, 128), jnp.float32),
            pltpu.SemaphoreType.DMA((2,)),
            pltpu.SemaphoreType.DMA((N_DEV - 1, CC)),
            pltpu.SemaphoreType.DMA((N_DEV - 1, CC)),
            pltpu.SemaphoreType.DMA((N_DEV - 1, CC)),
            pltpu.SemaphoreType.DMA((N_DEV - 1, CC)),
            pltpu.SemaphoreType.DMA((N_DEV - 1,)),
            pltpu.SemaphoreType.DMA((N_DEV - 1,)),
            pltpu.SemaphoreType.DMA((N_DEV - 1,)),
            pltpu.SemaphoreType.DMA((N_DEV - 1,)),
            pltpu.SemaphoreType.DMA((N_DEV - 1,)),
            pltpu.SemaphoreType.DMA((N_DEV - 1,)),
        ],
        compiler_params=pltpu.CompilerParams(
            collective_id=0,
            vmem_limit_bytes=50 * 1024 * 1024,
        ),
    )(x, w16)

    return (yc.astype(jnp.float32) * scale[0, 0]).astype(jnp.bfloat16)


# device time: 144088 ns/iter; 1.0177x vs baseline; 1.0177x over previous
import jax
import jax.numpy as jnp
from jax import lax
from jax.experimental import pallas as pl
from jax.experimental.pallas import tpu as pltpu

N_DEV = 4
M = 4096
N = 2048
NH = N // 2
CC = 2
CH = NH // CC
KS = 1024
MQ = M // N_DEV
CW, CCW = 0, 1


def _body(xb, w_ref, yc_ref, scale_ref,
          rcv_cw, rcv_ccw, sb_cw, sb_ccw, own_cw, own_ccw, amax_ref,
          rs_s_cw, rs_r_cw, rs_s_ccw, rs_r_ccw,
          ax_s, ax_r, ag_s_cw, ag_r_cw, ag_s_ccw, ag_r_ccw):
    my = lax.axis_index("i")
    right = lax.rem(my + 1, N_DEV)
    left = lax.rem(my + N_DEV - 1, N_DEV)

    barrier = pltpu.get_barrier_semaphore()
    for nbr in (left, right):
        pl.semaphore_signal(barrier, inc=1, device_id=(nbr,),
                            device_id_type=pl.DeviceIdType.MESH)
    pl.semaphore_wait(barrier, 2)

    def wslice(ring, c):
        base = ring * NH + c * CH
        return w_ref[:, base:base + CH]

    def rs_rdma(s, ring, c):
        rcv = rcv_cw if ring == CW else rcv_ccw
        sb = sb_cw if ring == CW else sb_ccw
        return pltpu.make_async_remote_copy(
            src_ref=sb.at[c] if s == 0 else rcv.at[s - 1, c],
            dst_ref=rcv.at[s, c],
            send_sem=(rs_s_cw if ring == CW else rs_s_ccw).at[s, c],
            recv_sem=(rs_r_cw if ring == CW else rs_r_ccw).at[s, c],
            device_id=(right,) if ring == CW else (left,),
            device_id_type=pl.DeviceIdType.MESH,
        )

    cur = {}
    for c in range(CC):
        sb_cw[c] = jnp.dot(xb[0], wslice(CW, c),
                           preferred_element_type=jnp.float32
                           ).astype(jnp.bfloat16)
        sb_ccw[c] = jnp.dot(xb[0], wslice(CCW, c),
                            preferred_element_type=jnp.float32
                            ).astype(jnp.bfloat16)
        for ring in (CW, CCW):
            d = rs_rdma(0, ring, c)
            d.start()
            cur[(ring, c)] = d

    am_parts = []
    for s in range(N_DEV - 1):
        nxt = {}
        for c in range(CC):
            for ring in (CW, CCW):
                blk = s + 1 if ring == CW else 3 - s
                part = jnp.dot(xb[blk], wslice(ring, c),
                               preferred_element_type=jnp.float32)
                cur[(ring, c)].wait()
                rcv = rcv_cw if ring == CW else rcv_ccw
                tot = rcv[s, c].astype(jnp.float32) + part
                if s < N_DEV - 2:
                    rcv[s, c] = tot.astype(jnp.bfloat16)
                    d = rs_rdma(s + 1, ring, c)
                    d.start()
                    nxt[(ring, c)] = d
                else:
                    own = own_cw if ring == CW else own_ccw
                    own[c] = tot
                    am_parts.append(jnp.max(tot))
        cur = nxt

    am = am_parts[0]
    for p in am_parts[1:]:
        am = jnp.maximum(am, p)
    am = jnp.maximum(am, 0.0)
    amax_ref[0, :, :] = jnp.full((8, 128), am, dtype=jnp.float32)
    opp = lax.rem(my + 2, N_DEV)
    ax_rdmas = []
    for j, (tgt, slot) in enumerate(((right, 3), (left, 1), (opp, 2))):
        rdma = pltpu.make_async_remote_copy(
            src_ref=amax_ref.at[0], dst_ref=amax_ref.at[slot],
            send_sem=ax_s.at[j], recv_sem=ax_r.at[slot - 1],
            device_id=(tgt,), device_id_type=pl.DeviceIdType.MESH)
        rdma.start()
        ax_rdmas.append(rdma)
    for rdma in ax_rdmas:
        rdma.wait()
    g_amax = jnp.max(amax_ref[:, :, :])
    scale = g_amax / 127.0
    inv_scale = 127.0 / g_amax
    scale_ref[0, 0] = scale

    qcw_own = lax.rem(my + 1, N_DEV)
    qccw_own = lax.rem(my + N_DEV - 1, N_DEV)
    for c in range(CC):
        qcw = jnp.clip(jnp.round(jnp.maximum(own_cw[c], 0.0) * inv_scale),
                       0.0, 127.0)
        qccw = jnp.clip(jnp.round(jnp.maximum(own_ccw[c], 0.0) * inv_scale),
                        0.0, 127.0)
        yc_ref[pl.ds(qcw_own * MQ, MQ),
               c * CH:(c + 1) * CH] = qcw.astype(jnp.int8)
        yc_ref[pl.ds(qccw_own * MQ, MQ),
               NH + c * CH:NH + (c + 1) * CH] = qccw.astype(jnp.int8)
    for t in range(N_DEV - 1):
        qcw_s = lax.rem(my + 1 - t + N_DEV, N_DEV)
        qccw_s = lax.rem(my + N_DEV - 1 + t, N_DEV)
        cw_slc = yc_ref.at[pl.ds(qcw_s * MQ, MQ), 0:NH]
        ccw_slc = yc_ref.at[pl.ds(qccw_s * MQ, MQ), NH:N]
        cw = pltpu.make_async_remote_copy(
            src_ref=cw_slc, dst_ref=cw_slc,
            send_sem=ag_s_cw.at[t], recv_sem=ag_r_cw.at[t],
            device_id=(right,), device_id_type=pl.DeviceIdType.MESH)
        ccw = pltpu.make_async_remote_copy(
            src_ref=ccw_slc, dst_ref=ccw_slc,
            send_sem=ag_s_ccw.at[t], recv_sem=ag_r_ccw.at[t],
            device_id=(left,), device_id_type=pl.DeviceIdType.MESH)
        cw.start()
        ccw.start()
        cw.wait()
        ccw.wait()


def kernel(x, w_mat):
    w16 = w_mat.astype(jnp.bfloat16)
    my = lax.axis_index("i")

    sidx = (my - jnp.arange(N_DEV)) % N_DEV
    xl = x.reshape(N_DEV, MQ, KS)[sidx].astype(jnp.bfloat16)

    yc, scale = pl.pallas_call(
        _body,
        out_shape=[
            jax.ShapeDtypeStruct((M, N), jnp.int8),
            jax.ShapeDtypeStruct((1, 1), jnp.float32),
        ],
        in_specs=[
            pl.BlockSpec(memory_space=pltpu.VMEM),
            pl.BlockSpec(memory_space=pltpu.VMEM),
        ],
        out_specs=[
            pl.BlockSpec(memory_space=pltpu.VMEM),
            pl.BlockSpec(memory_space=pltpu.SMEM),
        ],
        scratch_shapes=[
            pltpu.VMEM((N_DEV - 1, CC, MQ, CH), jnp.bfloat16),
            pltpu.VMEM((N_DEV - 1, CC, MQ, CH), jnp.bfloat16),
            pltpu.VMEM((CC, MQ, CH), jnp.bfloat16),
            pltpu.VMEM((CC, MQ, CH), jnp.bfloat16),
            pltpu.VMEM((CC, MQ, CH), jnp.float32),
            pltpu.VMEM((CC, MQ, CH), jnp.float32),
            pltpu.VMEM((N_DEV, 8, 128), jnp.float32),
            pltpu.SemaphoreType.DMA((N_DEV - 1, CC)),
            pltpu.SemaphoreType.DMA((N_DEV - 1, CC)),
            pltpu.SemaphoreType.DMA((N_DEV - 1, CC)),
            pltpu.SemaphoreType.DMA((N_DEV - 1, CC)),
            pltpu.SemaphoreType.DMA((N_DEV - 1,)),
            pltpu.SemaphoreType.DMA((N_DEV - 1,)),
            pltpu.SemaphoreType.DMA((N_DEV - 1,)),
            pltpu.SemaphoreType.DMA((N_DEV - 1,)),
            pltpu.SemaphoreType.DMA((N_DEV - 1,)),
            pltpu.SemaphoreType.DMA((N_DEV - 1,)),
        ],
        compiler_params=pltpu.CompilerParams(collective_id=0),
    )(xl, w16)

    return (yc.astype(jnp.float32) * scale[0, 0]).astype(jnp.bfloat16)
